# baseline (speedup 1.0000x reference)
import functools

import jax
import jax.numpy as jnp
import numpy as np
from jax import lax
from jax.experimental import pallas as pl
from jax.experimental.pallas import tpu as pltpu
from jax.experimental.pallas import tpu_sc as plsc

EMBED_DIM = 256
N_LEVELS = 4
N_HEADS = 8
N_POINTS = 4
HEAD_DIM = EMBED_DIM // N_HEADS
LP = N_LEVELS * N_POINTS
SHAPES = np.array([[48, 48], [24, 24], [12, 12], [6, 6]], dtype=np.int64)
LEN_IN = int((SHAPES[:, 0] * SHAPES[:, 1]).sum())
LQ_PAD = 3072
QBLK = 256
NCHUNK = LQ_PAD // 16
GRP = 8
NGRP = NCHUNK // GRP
PK_GRP = GRP * 64 * 16
TSTRIDE = HEAD_DIM + 1
OSTRIDE = HEAD_DIM + 1
OUT_GRP = GRP * 16 * OSTRIDE
TBL = LEN_IN * TSTRIDE
WBITS = 20
WSCALE = float((1 << WBITS) - 1)


def _lane_consts():
    lvl = np.tile(np.repeat(np.arange(N_LEVELS), N_POINTS), N_HEADS)
    W = SHAPES[lvl, 1].astype(np.float32)
    H = SHAPES[lvl, 0].astype(np.float32)
    areas = (SHAPES[:, 0] * SHAPES[:, 1]).astype(np.int64)
    start = np.concatenate([[0], np.cumsum(areas)[:-1]])
    ls = start[lvl].astype(np.int32)
    return (W.reshape(1, -1), H.reshape(1, -1),
            SHAPES[lvl, 1].astype(np.int32).reshape(1, -1), ls.reshape(1, -1))


def _sblk():
    s = np.zeros((N_HEADS * LP, N_HEADS * LP), np.float32)
    for m in range(N_HEADS):
        s[m * LP:(m + 1) * LP, m * LP:(m + 1) * LP] = 1.0
    return s


def _ref_bcast():
    lvl = np.tile(np.repeat(np.arange(N_LEVELS), N_POINTS), N_HEADS)
    bx = np.zeros((2 * N_LEVELS, 128), np.float32)
    by = np.zeros((2 * N_LEVELS, 128), np.float32)
    for k in range(128):
        bx[2 * lvl[k], k] = 1.0
        by[2 * lvl[k] + 1, k] = 1.0
    return bx, by


def _pack(r, w):
    ru = r.astype(jnp.uint32) << WBITS
    wq = (w * WSCALE + 0.5).astype(jnp.int32).astype(jnp.uint32)
    return ru | wq


def _prep_body(q_ref, inf_ref, rp_ref,
               wval_ref, bval_ref, wox_ref, box_ref, woy_ref, boy_ref,
               wat_ref, bat_ref, sblk_ref, bx_ref, by_ref,
               wf_ref, hf_ref, wi_ref, ls_ref,
               val_ref, p0_ref, p1_ref, p2_ref, p3_ref):
    q = q_ref[0]
    inf = inf_ref[0]
    val_ref[0] = jnp.dot(inf, wval_ref[...],
                         preferred_element_type=jnp.float32) + bval_ref[...]

    ox = jnp.dot(q, wox_ref[...], preferred_element_type=jnp.float32) + box_ref[...]
    oy = jnp.dot(q, woy_ref[...], preferred_element_type=jnp.float32) + boy_ref[...]
    lg = jnp.dot(q, wat_ref[...], preferred_element_type=jnp.float32) + bat_ref[...]
    lg = lg - jnp.max(lg, axis=1, keepdims=True)
    e = jnp.exp(lg)
    ssum = jnp.dot(e, sblk_ref[...], preferred_element_type=jnp.float32,
                   precision=lax.Precision.HIGHEST)
    attn = e / ssum

    rp = rp_ref[0]
    rx = jnp.dot(rp, bx_ref[...], preferred_element_type=jnp.float32,
                 precision=lax.Precision.HIGHEST)
    ry = jnp.dot(rp, by_ref[...], preferred_element_type=jnp.float32,
                 precision=lax.Precision.HIGHEST)

    wf = wf_ref[...]
    hf = hf_ref[...]
    px = rx * wf + ox - 0.5
    py = ry * hf + oy - 0.5
    x0 = jnp.floor(px)
    y0 = jnp.floor(py)
    fx = px - x0
    fy = py - y0
    vx0 = (x0 >= 0.0) & (x0 <= wf - 1.0)
    vx1 = (x0 >= -1.0) & (x0 <= wf - 2.0)
    vy0 = (y0 >= 0.0) & (y0 <= hf - 1.0)
    vy1 = (y0 >= -1.0) & (y0 <= hf - 2.0)
    wx0 = jnp.where(vx0, 1.0 - fx, 0.0)
    wx1 = jnp.where(vx1, fx, 0.0)
    wy0 = jnp.where(vy0, 1.0 - fy, 0.0)
    wy1 = jnp.where(vy1, fy, 0.0)

    xc0 = jnp.clip(x0, 0.0, wf - 1.0).astype(jnp.int32)
    xc1 = jnp.clip(x0 + 1.0, 0.0, wf - 1.0).astype(jnp.int32)
    yc0 = jnp.clip(y0, 0.0, hf - 1.0).astype(jnp.int32)
    yc1 = jnp.clip(y0 + 1.0, 0.0, hf - 1.0).astype(jnp.int32)

    wi = wi_ref[...]
    ls = ls_ref[...]
    r0 = ls + yc0 * wi
    r1 = ls + yc1 * wi
    p0_ref[0] = _pack(r0 + xc0, wx0 * wy0 * attn)
    p1_ref[0] = _pack(r0 + xc1, wx1 * wy0 * attn)
    p2_ref[0] = _pack(r1 + xc0, wx0 * wy1 * attn)
    p3_ref[0] = _pack(r1 + xc1, wx1 * wy1 * attn)


def _prep_call(q_p, inf_p, rp_p, consts):
    n_grid = q_p.shape[0]
    grid = (n_grid, LQ_PAD // QBLK)
    qspec = pl.BlockSpec((1, QBLK, EMBED_DIM), lambda n, i: (n, i, 0))
    lspec = pl.BlockSpec((1, QBLK, 128), lambda n, i: (n, i, 0))
    rspec = pl.BlockSpec((1, QBLK, 2 * N_LEVELS), lambda n, i: (n, i, 0))

    def full(shape):
        return pl.BlockSpec(shape, lambda n, i: tuple(0 for _ in shape))

    out_shapes = ([jax.ShapeDtypeStruct((n_grid, LQ_PAD, EMBED_DIM), jnp.float32)]
                  + [jax.ShapeDtypeStruct((n_grid, LQ_PAD, 128), jnp.uint32)] * 4)
    in_specs = [qspec, qspec, rspec,
                full((EMBED_DIM, EMBED_DIM)), full((1, EMBED_DIM)),
                full((EMBED_DIM, 128)), full((1, 128)),
                full((EMBED_DIM, 128)), full((1, 128)),
                full((EMBED_DIM, 128)), full((1, 128)),
                full((128, 128)),
                full((2 * N_LEVELS, 128)), full((2 * N_LEVELS, 128)),
                full((1, 128)), full((1, 128)), full((1, 128)), full((1, 128))]
    out_specs = [qspec] + [lspec] * 4
    return pl.pallas_call(
        _prep_body, grid=grid, in_specs=in_specs, out_specs=out_specs,
        out_shape=out_shapes)(q_p, inf_p, rp_p, *consts)


def _sc_body(val_hbm, pk_hbm, out_hbm, table_v, pk_v, out_v):
    c = lax.axis_index("c")
    s = lax.axis_index("s")
    wid = s * 2 + c
    n = wid // N_HEADS
    m = wid % N_HEADS
    pltpu.sync_copy(val_hbm.at[n, m], table_v)
    lane = jnp.arange(16, dtype=jnp.int32) * OSTRIDE

    def group_body(go, carry):
        pltpu.sync_copy(pk_hbm.at[n, m, go], pk_v)
        for g in range(GRP):
            for jh in range(2):
                def s_body(si, accs):
                    off = g * 1024 + si * 16
                    pk = pk_v[pl.ds(off, 16)]
                    r = (pk >> WBITS).astype(jnp.int32) * TSTRIDE + (jh * 16)
                    w = ((pk & ((1 << WBITS) - 1)).astype(jnp.int32)
                         .astype(jnp.float32)) * (1.0 / WSCALE)
                    return tuple(
                        accs[j] + plsc.load_gather(table_v, [r + j]) * w
                        for j in range(16))

                accs = lax.fori_loop(
                    0, 64, s_body,
                    tuple(jnp.zeros((16,), jnp.float32) for _ in range(16)))
                for j in range(16):
                    plsc.store_scatter(
                        out_v,
                        [lane + (g * 16 * OSTRIDE + jh * 16 + j)], accs[j])
        pltpu.sync_copy(out_v, out_hbm.at[n, m, go])
        return carry

    lax.fori_loop(0, NGRP, group_body, 0)


def _sc_call(val_t, pk_t):
    n_b = val_t.shape[0]
    mesh = plsc.VectorSubcoreMesh(core_axis_name="c", subcore_axis_name="s")
    kfn = pl.kernel(
        _sc_body,
        out_type=jax.ShapeDtypeStruct((n_b, N_HEADS, NGRP, OUT_GRP), jnp.float32),
        mesh=mesh,
        compiler_params=pltpu.CompilerParams(needs_layout_passes=False),
        scratch_types=[
            pltpu.VMEM((TBL,), jnp.float32),
            pltpu.VMEM((PK_GRP,), jnp.uint32),
            pltpu.VMEM((OUT_GRP,), jnp.float32),
        ],
    )
    return kfn(val_t, pk_t)


def _out_body(smp_ref, wout_ref, bout_ref, out_ref):
    acc = jnp.broadcast_to(bout_ref[...], (QBLK, EMBED_DIM))
    for m in range(N_HEADS):
        acc = acc + jnp.dot(smp_ref[0, m, :, 0:HEAD_DIM], wout_ref[m],
                            preferred_element_type=jnp.float32)
    out_ref[0] = acc


def _out_call(smp, wout_r, bout):
    n_b = smp.shape[0]
    grid = (n_b, LQ_PAD // QBLK)
    return pl.pallas_call(
        _out_body, grid=grid,
        in_specs=[pl.BlockSpec((1, N_HEADS, QBLK, OSTRIDE),
                               lambda n, i: (n, 0, i, 0)),
                  pl.BlockSpec((N_HEADS, HEAD_DIM, EMBED_DIM),
                               lambda n, i: (0, 0, 0)),
                  pl.BlockSpec((1, EMBED_DIM), lambda n, i: (0, 0))],
        out_specs=pl.BlockSpec((1, QBLK, EMBED_DIM), lambda n, i: (n, i, 0)),
        out_shape=jax.ShapeDtypeStruct((n_b, LQ_PAD, EMBED_DIM), jnp.float32),
    )(smp, wout_r, bout)


def _to_sc_layout(parts):
    n_b = parts[0].shape[0]
    t = jnp.stack(parts, axis=-1)
    t = t.reshape(n_b, LQ_PAD, N_HEADS, 64)
    t = t.transpose(0, 2, 1, 3)
    t = t.reshape(n_b, N_HEADS, NCHUNK, 16, 64)
    t = t.transpose(0, 1, 2, 4, 3)
    return t.reshape(n_b, N_HEADS, NGRP, PK_GRP)


@jax.jit
def kernel(query, reference_points, input_flatten, input_spatial_shapes,
           input_level_start_index, W_off, b_off, W_attn, b_attn,
           W_val, b_val, W_out, b_out):
    n_b, lq, _ = query.shape
    pad = LQ_PAD - lq
    q_p = jnp.pad(query, ((0, 0), (0, pad), (0, 0)))
    inf_p = jnp.pad(input_flatten, ((0, 0), (0, pad), (0, 0)))
    rp_p = jnp.pad(reference_points.reshape(n_b, lq, 2 * N_LEVELS),
                   ((0, 0), (0, pad), (0, 0)))

    wo = W_off.reshape(EMBED_DIM, 128, 2)
    bo = b_off.reshape(1, 128, 2)
    wf, hf, wi, ls = _lane_consts()
    bx, by = _ref_bcast()
    consts = (W_val, b_val.reshape(1, EMBED_DIM),
              wo[:, :, 0], bo[:, :, 0], wo[:, :, 1], bo[:, :, 1],
              W_attn, b_attn.reshape(1, 128), jnp.asarray(_sblk()),
              jnp.asarray(bx), jnp.asarray(by),
              jnp.asarray(wf), jnp.asarray(hf), jnp.asarray(wi),
              jnp.asarray(ls))

    val, p0, p1, p2, p3 = _prep_call(q_p, inf_p, rp_p, consts)
    pk_t = _to_sc_layout([p0, p1, p2, p3])

    val_t = val[:, :LEN_IN].reshape(n_b, LEN_IN, N_HEADS, HEAD_DIM)
    val_t = val_t.transpose(0, 2, 1, 3)
    val_t = jnp.pad(val_t, ((0, 0), (0, 0), (0, 0), (0, TSTRIDE - HEAD_DIM)))
    val_t = val_t.reshape(n_b, N_HEADS, TBL)

    smp = _sc_call(val_t, pk_t)
    smp = smp.reshape(n_b, N_HEADS, LQ_PAD, OSTRIDE)

    out = _out_call(smp, W_out.reshape(N_HEADS, HEAD_DIM, EMBED_DIM),
                    b_out.reshape(1, EMBED_DIM))
    return out[:, :lq]


# speedup vs baseline: 1.1143x; 1.1143x over previous
import functools

import jax
import jax.numpy as jnp
import numpy as np
from jax import lax
from jax.experimental import pallas as pl
from jax.experimental.pallas import tpu as pltpu
from jax.experimental.pallas import tpu_sc as plsc

EMBED_DIM = 256
N_LEVELS = 4
N_HEADS = 8
N_POINTS = 4
HEAD_DIM = EMBED_DIM // N_HEADS
LP = N_LEVELS * N_POINTS
SHAPES = np.array([[48, 48], [24, 24], [12, 12], [6, 6]], dtype=np.int64)
LEN_IN = int((SHAPES[:, 0] * SHAPES[:, 1]).sum())
LQ_PAD = 3072
QBLK = 256
NCHUNK = LQ_PAD // 16
GRP = 8
NGRP = NCHUNK // GRP
PK_GRP = GRP * 64 * 16
TSTRIDE = HEAD_DIM + 1
OSTRIDE = HEAD_DIM + 1
OUT_GRP = GRP * 16 * OSTRIDE
TBL = LEN_IN * TSTRIDE
WBITS = 20
WSCALE = float((1 << WBITS) - 1)


def _lane_consts():
    lvl = np.tile(np.repeat(np.arange(N_LEVELS), N_POINTS), N_HEADS)
    W = SHAPES[lvl, 1].astype(np.float32)
    H = SHAPES[lvl, 0].astype(np.float32)
    areas = (SHAPES[:, 0] * SHAPES[:, 1]).astype(np.int64)
    start = np.concatenate([[0], np.cumsum(areas)[:-1]])
    ls = start[lvl].astype(np.int32)
    return (W.reshape(1, -1), H.reshape(1, -1),
            SHAPES[lvl, 1].astype(np.int32).reshape(1, -1), ls.reshape(1, -1))


def _sblk():
    s = np.zeros((N_HEADS * LP, N_HEADS * LP), np.float32)
    for m in range(N_HEADS):
        s[m * LP:(m + 1) * LP, m * LP:(m + 1) * LP] = 1.0
    return s


def _ref_bcast():
    lvl = np.tile(np.repeat(np.arange(N_LEVELS), N_POINTS), N_HEADS)
    bx = np.zeros((2 * N_LEVELS, 128), np.float32)
    by = np.zeros((2 * N_LEVELS, 128), np.float32)
    for k in range(128):
        bx[2 * lvl[k], k] = 1.0
        by[2 * lvl[k] + 1, k] = 1.0
    return bx, by


def _pack(r, w):
    ru = r.astype(jnp.uint32) << WBITS
    wq = (w * WSCALE + 0.5).astype(jnp.int32).astype(jnp.uint32)
    return ru | wq


def _prep_body(q_ref, inf_ref, rp_ref,
               wval_ref, bval_ref, wox_ref, box_ref, woy_ref, boy_ref,
               wat_ref, bat_ref, sblk_ref, bx_ref, by_ref,
               wf_ref, hf_ref, wi_ref, ls_ref,
               val_ref, p0_ref, p1_ref, p2_ref, p3_ref):
    q = q_ref[0]
    inf = inf_ref[0]
    val_ref[0] = jnp.dot(inf, wval_ref[...],
                         preferred_element_type=jnp.float32) + bval_ref[...]

    ox = jnp.dot(q, wox_ref[...], preferred_element_type=jnp.float32) + box_ref[...]
    oy = jnp.dot(q, woy_ref[...], preferred_element_type=jnp.float32) + boy_ref[...]
    lg = jnp.dot(q, wat_ref[...], preferred_element_type=jnp.float32) + bat_ref[...]
    lg = lg - jnp.max(lg, axis=1, keepdims=True)
    e = jnp.exp(lg)
    ssum = jnp.dot(e, sblk_ref[...], preferred_element_type=jnp.float32,
                   precision=lax.Precision.HIGHEST)
    attn = e / ssum

    rp = rp_ref[0]
    rx = jnp.dot(rp, bx_ref[...], preferred_element_type=jnp.float32,
                 precision=lax.Precision.HIGHEST)
    ry = jnp.dot(rp, by_ref[...], preferred_element_type=jnp.float32,
                 precision=lax.Precision.HIGHEST)

    wf = wf_ref[...]
    hf = hf_ref[...]
    px = rx * wf + ox - 0.5
    py = ry * hf + oy - 0.5
    x0 = jnp.floor(px)
    y0 = jnp.floor(py)
    fx = px - x0
    fy = py - y0
    vx0 = (x0 >= 0.0) & (x0 <= wf - 1.0)
    vx1 = (x0 >= -1.0) & (x0 <= wf - 2.0)
    vy0 = (y0 >= 0.0) & (y0 <= hf - 1.0)
    vy1 = (y0 >= -1.0) & (y0 <= hf - 2.0)
    wx0 = jnp.where(vx0, 1.0 - fx, 0.0)
    wx1 = jnp.where(vx1, fx, 0.0)
    wy0 = jnp.where(vy0, 1.0 - fy, 0.0)
    wy1 = jnp.where(vy1, fy, 0.0)

    xc0 = jnp.clip(x0, 0.0, wf - 1.0).astype(jnp.int32)
    xc1 = jnp.clip(x0 + 1.0, 0.0, wf - 1.0).astype(jnp.int32)
    yc0 = jnp.clip(y0, 0.0, hf - 1.0).astype(jnp.int32)
    yc1 = jnp.clip(y0 + 1.0, 0.0, hf - 1.0).astype(jnp.int32)

    wi = wi_ref[...]
    ls = ls_ref[...]
    r0 = ls + yc0 * wi
    r1 = ls + yc1 * wi
    rowid = (jax.lax.broadcasted_iota(jnp.int32, (QBLK, 128), 0)
             + pl.program_id(1) * QBLK)
    live = rowid < LEN_IN
    zero = jnp.zeros((QBLK, 128), jnp.uint32)
    p0_ref[0] = jnp.where(live, _pack(r0 + xc0, wx0 * wy0 * attn), zero)
    p1_ref[0] = jnp.where(live, _pack(r0 + xc1, wx1 * wy0 * attn), zero)
    p2_ref[0] = jnp.where(live, _pack(r1 + xc0, wx0 * wy1 * attn), zero)
    p3_ref[0] = jnp.where(live, _pack(r1 + xc1, wx1 * wy1 * attn), zero)


def _prep_call(q_p, inf_p, rp_p, consts):
    n_grid = q_p.shape[0]
    grid = (n_grid, LQ_PAD // QBLK)
    qspec = pl.BlockSpec((1, QBLK, EMBED_DIM), lambda n, i: (n, i, 0))
    lspec = pl.BlockSpec((1, QBLK, 128), lambda n, i: (n, i, 0))
    rspec = pl.BlockSpec((1, QBLK, 2 * N_LEVELS), lambda n, i: (n, i, 0))

    def full(shape):
        return pl.BlockSpec(shape, lambda n, i: tuple(0 for _ in shape))

    out_shapes = ([jax.ShapeDtypeStruct((n_grid, LQ_PAD, EMBED_DIM), jnp.float32)]
                  + [jax.ShapeDtypeStruct((n_grid, LQ_PAD, 128), jnp.uint32)] * 4)
    in_specs = [qspec, qspec, rspec,
                full((EMBED_DIM, EMBED_DIM)), full((1, EMBED_DIM)),
                full((EMBED_DIM, 128)), full((1, 128)),
                full((EMBED_DIM, 128)), full((1, 128)),
                full((EMBED_DIM, 128)), full((1, 128)),
                full((128, 128)),
                full((2 * N_LEVELS, 128)), full((2 * N_LEVELS, 128)),
                full((1, 128)), full((1, 128)), full((1, 128)), full((1, 128))]
    out_specs = [qspec] + [lspec] * 4
    return pl.pallas_call(
        _prep_body, grid=grid, in_specs=in_specs, out_specs=out_specs,
        out_shape=out_shapes)(q_p, inf_p, rp_p, *consts)


def _sc_body(val_hbm, pk_hbm, out_hbm, table_v,
             pk_v0, pk_v1, out_v0, out_v1, sin0, sin1, sout0, sout1):
    c = lax.axis_index("c")
    s = lax.axis_index("s")
    wid = s * 2 + c
    n = wid // N_HEADS
    m = wid % N_HEADS
    pltpu.sync_copy(val_hbm.at[n, m], table_v)
    lane = jnp.arange(16, dtype=jnp.int32) * OSTRIDE

    pk_bufs = (pk_v0, pk_v1)
    out_bufs = (out_v0, out_v1)
    sins = (sin0, sin1)
    souts = (sout0, sout1)
    pltpu.async_copy(pk_hbm.at[n, m, 0], pk_v0, sin0)

    def go2_body(go2, carry):
        for half in range(2):
            go = go2 * 2 + half
            pk_v = pk_bufs[half]
            out_v = out_bufs[half]
            pltpu.make_async_copy(pk_hbm.at[n, m, go], pk_v,
                                  sins[half]).wait()

            @pl.when(go < NGRP - 1)
            def _():
                pltpu.async_copy(pk_hbm.at[n, m, go + 1],
                                 pk_bufs[1 - half], sins[1 - half])

            @pl.when(go2 > 0)
            def _():
                pltpu.make_async_copy(out_v, out_hbm.at[n, m, go - 2],
                                      souts[half]).wait()

            for g in range(GRP):
                for jh in range(2):
                    def s_body(si, accs):
                        off = g * 1024 + si * 16
                        pk = pk_v[pl.ds(off, 16)]
                        r = ((pk >> WBITS).astype(jnp.int32) * TSTRIDE
                             + (jh * 16))
                        w = ((pk & ((1 << WBITS) - 1)).astype(jnp.int32)
                             .astype(jnp.float32)) * (1.0 / WSCALE)
                        return tuple(
                            accs[j] + plsc.load_gather(table_v, [r + j]) * w
                            for j in range(16))

                    accs = lax.fori_loop(
                        0, 64, s_body,
                        tuple(jnp.zeros((16,), jnp.float32)
                              for _ in range(16)))
                    for j in range(16):
                        plsc.store_scatter(
                            out_v,
                            [lane + (g * 16 * OSTRIDE + jh * 16 + j)],
                            accs[j])
            pltpu.async_copy(out_v, out_hbm.at[n, m, go], souts[half])
        return carry

    lax.fori_loop(0, NGRP // 2, go2_body, 0)
    pltpu.make_async_copy(out_v0, out_hbm.at[n, m, NGRP - 2], sout0).wait()
    pltpu.make_async_copy(out_v1, out_hbm.at[n, m, NGRP - 1], sout1).wait()


def _sc_call(val_t, pk_t):
    n_b = val_t.shape[0]
    mesh = plsc.VectorSubcoreMesh(core_axis_name="c", subcore_axis_name="s")
    kfn = pl.kernel(
        _sc_body,
        out_type=jax.ShapeDtypeStruct((n_b, N_HEADS, NGRP, OUT_GRP), jnp.float32),
        mesh=mesh,
        compiler_params=pltpu.CompilerParams(needs_layout_passes=False),
        scratch_types=[
            pltpu.VMEM((TBL,), jnp.float32),
            pltpu.VMEM((PK_GRP,), jnp.uint32),
            pltpu.VMEM((PK_GRP,), jnp.uint32),
            pltpu.VMEM((OUT_GRP,), jnp.float32),
            pltpu.VMEM((OUT_GRP,), jnp.float32),
            pltpu.SemaphoreType.DMA,
            pltpu.SemaphoreType.DMA,
            pltpu.SemaphoreType.DMA,
            pltpu.SemaphoreType.DMA,
        ],
    )
    return kfn(val_t, pk_t)


def _out_body(smp_ref, wout_ref, bout_ref, out_ref):
    acc = jnp.broadcast_to(bout_ref[...], (QBLK, EMBED_DIM))
    for m in range(N_HEADS):
        acc = acc + jnp.dot(smp_ref[0, m, :, 0:HEAD_DIM], wout_ref[m],
                            preferred_element_type=jnp.float32)
    out_ref[0] = acc


def _out_call(smp, wout_r, bout, lq):
    n_b = smp.shape[0]
    grid = (n_b, LQ_PAD // QBLK)
    return pl.pallas_call(
        _out_body, grid=grid,
        in_specs=[pl.BlockSpec((1, N_HEADS, QBLK, OSTRIDE),
                               lambda n, i: (n, 0, i, 0)),
                  pl.BlockSpec((N_HEADS, HEAD_DIM, EMBED_DIM),
                               lambda n, i: (0, 0, 0)),
                  pl.BlockSpec((1, EMBED_DIM), lambda n, i: (0, 0))],
        out_specs=pl.BlockSpec((1, QBLK, EMBED_DIM), lambda n, i: (n, i, 0)),
        out_shape=jax.ShapeDtypeStruct((n_b, lq, EMBED_DIM), jnp.float32),
    )(smp, wout_r, bout)


def _to_sc_layout(parts):
    n_b = parts[0].shape[0]
    t = jnp.stack(parts, axis=-1)
    t = t.reshape(n_b, LQ_PAD, N_HEADS, 64)
    t = t.transpose(0, 2, 1, 3)
    t = t.reshape(n_b, N_HEADS, NCHUNK, 16, 64)
    t = t.transpose(0, 1, 2, 4, 3)
    return t.reshape(n_b, N_HEADS, NGRP, PK_GRP)


@jax.jit
def kernel(query, reference_points, input_flatten, input_spatial_shapes,
           input_level_start_index, W_off, b_off, W_attn, b_attn,
           W_val, b_val, W_out, b_out):
    n_b, lq, _ = query.shape
    q_p = query
    inf_p = input_flatten
    rp_p = reference_points.reshape(n_b, lq, 2 * N_LEVELS)

    wo = W_off.reshape(EMBED_DIM, 128, 2)
    bo = b_off.reshape(1, 128, 2)
    wf, hf, wi, ls = _lane_consts()
    bx, by = _ref_bcast()
    consts = (W_val, b_val.reshape(1, EMBED_DIM),
              wo[:, :, 0], bo[:, :, 0], wo[:, :, 1], bo[:, :, 1],
              W_attn, b_attn.reshape(1, 128), jnp.asarray(_sblk()),
              jnp.asarray(bx), jnp.asarray(by),
              jnp.asarray(wf), jnp.asarray(hf), jnp.asarray(wi),
              jnp.asarray(ls))

    val, p0, p1, p2, p3 = _prep_call(q_p, inf_p, rp_p, consts)
    pk_t = _to_sc_layout([p0, p1, p2, p3])

    val_t = val[:, :LEN_IN].reshape(n_b, LEN_IN, N_HEADS, HEAD_DIM)
    val_t = val_t.transpose(0, 2, 1, 3)
    val_t = jnp.pad(val_t, ((0, 0), (0, 0), (0, 0), (0, TSTRIDE - HEAD_DIM)))
    val_t = val_t.reshape(n_b, N_HEADS, TBL)

    smp = _sc_call(val_t, pk_t)
    smp = smp.reshape(n_b, N_HEADS, LQ_PAD, OSTRIDE)

    return _out_call(smp, W_out.reshape(N_HEADS, HEAD_DIM, EMBED_DIM),
                     b_out.reshape(1, EMBED_DIM), lq)


# speedup vs baseline: 1.5142x; 1.3588x over previous
import functools

import jax
import jax.numpy as jnp
import numpy as np
from jax import lax
from jax.experimental import pallas as pl
from jax.experimental.pallas import tpu as pltpu
from jax.experimental.pallas import tpu_sc as plsc

EMBED_DIM = 256
N_LEVELS = 4
N_HEADS = 8
N_POINTS = 4
HEAD_DIM = EMBED_DIM // N_HEADS
LP = N_LEVELS * N_POINTS
SHAPES = np.array([[48, 48], [24, 24], [12, 12], [6, 6]], dtype=np.int64)
LEN_IN = int((SHAPES[:, 0] * SHAPES[:, 1]).sum())
LQ_PAD = 3072
QBLK = 256
NCHUNK = LQ_PAD // 16
GRP = 8
NGRP = NCHUNK // GRP
PK_GRP = GRP * 64 * 16
TSTRIDE = HEAD_DIM + 1
OSTRIDE = HEAD_DIM + 1
OUT_GRP = GRP * 16 * OSTRIDE
TBL = LEN_IN * TSTRIDE
WBITS = 20
WSCALE = float((1 << WBITS) - 1)


def _lane_consts():
    lvl = np.tile(np.repeat(np.arange(N_LEVELS), N_POINTS), N_HEADS)
    W = SHAPES[lvl, 1].astype(np.float32)
    H = SHAPES[lvl, 0].astype(np.float32)
    areas = (SHAPES[:, 0] * SHAPES[:, 1]).astype(np.int64)
    start = np.concatenate([[0], np.cumsum(areas)[:-1]])
    ls = start[lvl].astype(np.int32)
    return (W.reshape(1, -1), H.reshape(1, -1),
            SHAPES[lvl, 1].astype(np.int32).reshape(1, -1), ls.reshape(1, -1))


def _sblk():
    s = np.zeros((N_HEADS * LP, N_HEADS * LP), np.float32)
    for m in range(N_HEADS):
        s[m * LP:(m + 1) * LP, m * LP:(m + 1) * LP] = 1.0
    return s


def _ref_bcast():
    lvl = np.tile(np.repeat(np.arange(N_LEVELS), N_POINTS), N_HEADS)
    bx = np.zeros((2 * N_LEVELS, 128), np.float32)
    by = np.zeros((2 * N_LEVELS, 128), np.float32)
    for k in range(128):
        bx[2 * lvl[k], k] = 1.0
        by[2 * lvl[k] + 1, k] = 1.0
    return bx, by


def _pack(r, w):
    ru = r.astype(jnp.uint32) << WBITS
    wq = (w * WSCALE + 0.5).astype(jnp.int32).astype(jnp.uint32)
    return ru | wq


def _prep_body(q_ref, inf_ref, rp_ref,
               wval_ref, bval_ref, wox_ref, box_ref, woy_ref, boy_ref,
               wat_ref, bat_ref, sblk_ref, bx_ref, by_ref,
               wf_ref, hf_ref, wi_ref, ls_ref,
               val_ref, p0_ref, p1_ref, p2_ref, p3_ref):
    q = q_ref[0]
    inf = inf_ref[0]
    val_ref[0] = jnp.dot(inf, wval_ref[...],
                         preferred_element_type=jnp.float32) + bval_ref[...]

    ox = jnp.dot(q, wox_ref[...], preferred_element_type=jnp.float32) + box_ref[...]
    oy = jnp.dot(q, woy_ref[...], preferred_element_type=jnp.float32) + boy_ref[...]
    lg = jnp.dot(q, wat_ref[...], preferred_element_type=jnp.float32) + bat_ref[...]
    lg = lg - jnp.max(lg, axis=1, keepdims=True)
    e = jnp.exp(lg)
    ssum = jnp.dot(e, sblk_ref[...], preferred_element_type=jnp.float32,
                   precision=lax.Precision.HIGHEST)
    attn = e / ssum

    rp = rp_ref[0]
    rx = jnp.dot(rp, bx_ref[...], preferred_element_type=jnp.float32,
                 precision=lax.Precision.HIGHEST)
    ry = jnp.dot(rp, by_ref[...], preferred_element_type=jnp.float32,
                 precision=lax.Precision.HIGHEST)

    wf = wf_ref[...]
    hf = hf_ref[...]
    px = rx * wf + ox - 0.5
    py = ry * hf + oy - 0.5
    x0 = jnp.floor(px)
    y0 = jnp.floor(py)
    fx = px - x0
    fy = py - y0
    vx0 = (x0 >= 0.0) & (x0 <= wf - 1.0)
    vx1 = (x0 >= -1.0) & (x0 <= wf - 2.0)
    vy0 = (y0 >= 0.0) & (y0 <= hf - 1.0)
    vy1 = (y0 >= -1.0) & (y0 <= hf - 2.0)
    wx0 = jnp.where(vx0, 1.0 - fx, 0.0)
    wx1 = jnp.where(vx1, fx, 0.0)
    wy0 = jnp.where(vy0, 1.0 - fy, 0.0)
    wy1 = jnp.where(vy1, fy, 0.0)

    xc0 = jnp.clip(x0, 0.0, wf - 1.0).astype(jnp.int32)
    xc1 = jnp.clip(x0 + 1.0, 0.0, wf - 1.0).astype(jnp.int32)
    yc0 = jnp.clip(y0, 0.0, hf - 1.0).astype(jnp.int32)
    yc1 = jnp.clip(y0 + 1.0, 0.0, hf - 1.0).astype(jnp.int32)

    wi = wi_ref[...]
    ls = ls_ref[...]
    r0 = ls + yc0 * wi
    r1 = ls + yc1 * wi
    rowid = (jax.lax.broadcasted_iota(jnp.int32, (QBLK, 128), 0)
             + pl.program_id(1) * QBLK)
    live = rowid < LEN_IN
    zero = jnp.zeros((QBLK, 128), jnp.uint32)
    p0_ref[0] = jnp.where(live, _pack(r0 + xc0, wx0 * wy0 * attn), zero).T
    p1_ref[0] = jnp.where(live, _pack(r0 + xc1, wx1 * wy0 * attn), zero).T
    p2_ref[0] = jnp.where(live, _pack(r1 + xc0, wx0 * wy1 * attn), zero).T
    p3_ref[0] = jnp.where(live, _pack(r1 + xc1, wx1 * wy1 * attn), zero).T


def _prep_call(q_p, inf_p, rp_p, consts):
    n_grid = q_p.shape[0]
    grid = (n_grid, LQ_PAD // QBLK)
    qspec = pl.BlockSpec((1, QBLK, EMBED_DIM), lambda n, i: (n, i, 0))
    lspec = pl.BlockSpec((1, QBLK, 128), lambda n, i: (n, i, 0))
    rspec = pl.BlockSpec((1, QBLK, 2 * N_LEVELS), lambda n, i: (n, i, 0))

    def full(shape):
        return pl.BlockSpec(shape, lambda n, i: tuple(0 for _ in shape))

    out_shapes = ([jax.ShapeDtypeStruct((n_grid, LQ_PAD, EMBED_DIM), jnp.float32)]
                  + [jax.ShapeDtypeStruct((n_grid, 128, LQ_PAD), jnp.uint32)] * 4)
    in_specs = [qspec, qspec, rspec,
                full((EMBED_DIM, EMBED_DIM)), full((1, EMBED_DIM)),
                full((EMBED_DIM, 128)), full((1, 128)),
                full((EMBED_DIM, 128)), full((1, 128)),
                full((EMBED_DIM, 128)), full((1, 128)),
                full((128, 128)),
                full((2 * N_LEVELS, 128)), full((2 * N_LEVELS, 128)),
                full((1, 128)), full((1, 128)), full((1, 128)), full((1, 128))]
    tspec = pl.BlockSpec((1, 128, QBLK), lambda n, i: (n, 0, i))
    out_specs = [qspec] + [tspec] * 4
    return pl.pallas_call(
        _prep_body, grid=grid, in_specs=in_specs, out_specs=out_specs,
        out_shape=out_shapes)(q_p, inf_p, rp_p, *consts)


def _sc_body(val_hbm, p0_hbm, p1_hbm, p2_hbm, p3_hbm, out_hbm, table_v,
             pk_v0, pk_v1, out_v0, out_v1, sin0, sin1, sout0, sout1):
    c = lax.axis_index("c")
    s = lax.axis_index("s")
    wid = s * 2 + c
    n = wid // N_HEADS
    m = wid % N_HEADS
    pltpu.sync_copy(val_hbm.at[n, m], table_v)
    lane = jnp.arange(16, dtype=jnp.int32) * OSTRIDE

    pk_bufs = (pk_v0, pk_v1)
    out_bufs = (out_v0, out_v1)
    sins = (sin0, sin1)
    souts = (sout0, sout1)
    pks = (p0_hbm, p1_hbm, p2_hbm, p3_hbm)

    def in_descs(go, buf, sem):
        return [pltpu.make_async_copy(
            pks[ci].at[n, pl.ds(m * 16, 16), pl.ds(go * 128, 128)],
            buf.at[pl.ds(ci * 16, 16)], sem) for ci in range(4)]

    def start_in(go, buf, sem):
        for d in in_descs(go, buf, sem):
            d.start()

    start_in(0, pk_v0, sin0)

    def go2_body(go2, carry):
        for half in range(2):
            go = go2 * 2 + half
            pk_v = pk_bufs[half]
            out_v = out_bufs[half]
            for d in in_descs(go, pk_v, sins[half]):
                d.wait()

            @pl.when(go < NGRP - 1)
            def _():
                start_in(go + 1, pk_bufs[1 - half], sins[1 - half])

            @pl.when(go2 > 0)
            def _():
                pltpu.make_async_copy(out_v, out_hbm.at[n, m, go - 2],
                                      souts[half]).wait()

            for g in range(GRP):
                for jh in range(2):
                    def s_body(si, accs):
                        pk = pk_v[si, pl.ds(g * 16, 16)]
                        r = ((pk >> WBITS).astype(jnp.int32) * TSTRIDE
                             + (jh * 16))
                        w = ((pk & ((1 << WBITS) - 1)).astype(jnp.int32)
                             .astype(jnp.float32)) * (1.0 / WSCALE)
                        return tuple(
                            accs[j] + plsc.load_gather(table_v, [r + j]) * w
                            for j in range(16))

                    accs = lax.fori_loop(
                        0, 64, s_body,
                        tuple(jnp.zeros((16,), jnp.float32)
                              for _ in range(16)))
                    for j in range(16):
                        plsc.store_scatter(
                            out_v,
                            [lane + (g * 16 * OSTRIDE + jh * 16 + j)],
                            accs[j])
            pltpu.async_copy(out_v, out_hbm.at[n, m, go], souts[half])
        return carry

    lax.fori_loop(0, NGRP // 2, go2_body, 0)
    pltpu.make_async_copy(out_v0, out_hbm.at[n, m, NGRP - 2], sout0).wait()
    pltpu.make_async_copy(out_v1, out_hbm.at[n, m, NGRP - 1], sout1).wait()


def _sc_call(val_t, p0, p1, p2, p3):
    n_b = val_t.shape[0]
    mesh = plsc.VectorSubcoreMesh(core_axis_name="c", subcore_axis_name="s")
    kfn = pl.kernel(
        _sc_body,
        out_type=jax.ShapeDtypeStruct((n_b, N_HEADS, NGRP, OUT_GRP), jnp.float32),
        mesh=mesh,
        compiler_params=pltpu.CompilerParams(needs_layout_passes=False),
        scratch_types=[
            pltpu.VMEM((TBL,), jnp.float32),
            pltpu.VMEM((64, 128), jnp.uint32),
            pltpu.VMEM((64, 128), jnp.uint32),
            pltpu.VMEM((OUT_GRP,), jnp.float32),
            pltpu.VMEM((OUT_GRP,), jnp.float32),
            pltpu.SemaphoreType.DMA,
            pltpu.SemaphoreType.DMA,
            pltpu.SemaphoreType.DMA,
            pltpu.SemaphoreType.DMA,
        ],
    )
    return kfn(val_t, p0, p1, p2, p3)


def _out_body(smp_ref, wout_ref, bout_ref, out_ref):
    acc = jnp.broadcast_to(bout_ref[...], (QBLK, EMBED_DIM))
    for m in range(N_HEADS):
        acc = acc + jnp.dot(smp_ref[0, m, :, 0:HEAD_DIM], wout_ref[m],
                            preferred_element_type=jnp.float32)
    out_ref[0] = acc


def _out_call(smp, wout_r, bout, lq):
    n_b = smp.shape[0]
    grid = (n_b, LQ_PAD // QBLK)
    return pl.pallas_call(
        _out_body, grid=grid,
        in_specs=[pl.BlockSpec((1, N_HEADS, QBLK, OSTRIDE),
                               lambda n, i: (n, 0, i, 0)),
                  pl.BlockSpec((N_HEADS, HEAD_DIM, EMBED_DIM),
                               lambda n, i: (0, 0, 0)),
                  pl.BlockSpec((1, EMBED_DIM), lambda n, i: (0, 0))],
        out_specs=pl.BlockSpec((1, QBLK, EMBED_DIM), lambda n, i: (n, i, 0)),
        out_shape=jax.ShapeDtypeStruct((n_b, lq, EMBED_DIM), jnp.float32),
    )(smp, wout_r, bout)


@jax.jit
def kernel(query, reference_points, input_flatten, input_spatial_shapes,
           input_level_start_index, W_off, b_off, W_attn, b_attn,
           W_val, b_val, W_out, b_out):
    n_b, lq, _ = query.shape
    q_p = query
    inf_p = input_flatten
    rp_p = reference_points.reshape(n_b, lq, 2 * N_LEVELS)

    wo = W_off.reshape(EMBED_DIM, 128, 2)
    bo = b_off.reshape(1, 128, 2)
    wf, hf, wi, ls = _lane_consts()
    bx, by = _ref_bcast()
    consts = (W_val, b_val.reshape(1, EMBED_DIM),
              wo[:, :, 0], bo[:, :, 0], wo[:, :, 1], bo[:, :, 1],
              W_attn, b_attn.reshape(1, 128), jnp.asarray(_sblk()),
              jnp.asarray(bx), jnp.asarray(by),
              jnp.asarray(wf), jnp.asarray(hf), jnp.asarray(wi),
              jnp.asarray(ls))

    val, p0, p1, p2, p3 = _prep_call(q_p, inf_p, rp_p, consts)

    val_t = val[:, :LEN_IN].reshape(n_b, LEN_IN, N_HEADS, HEAD_DIM)
    val_t = val_t.transpose(0, 2, 1, 3)
    val_t = jnp.pad(val_t, ((0, 0), (0, 0), (0, 0), (0, TSTRIDE - HEAD_DIM)))
    val_t = val_t.reshape(n_b, N_HEADS, TBL)

    smp = _sc_call(val_t, p0, p1, p2, p3)
    smp = smp.reshape(n_b, N_HEADS, LQ_PAD, OSTRIDE)

    return _out_call(smp, W_out.reshape(N_HEADS, HEAD_DIM, EMBED_DIM),
                     b_out.reshape(1, EMBED_DIM), lq)
